# baseline (speedup 1.0000x reference)
import functools

import jax
import jax.numpy as jnp
from jax import lax
from jax.experimental import pallas as pl
from jax.experimental.pallas import tpu as pltpu
from jax.experimental.pallas import tpu_sc as plsc

DIM = 33
NC, NS, L = 2, 16, 16
NW = NC * NS
B, H, W = 8, 512, 512
TR, TCC = 8, 128
TPB = (H // TR) * (W // TCC)
TPW = TPB // NW
NT = B * TPW
CBLK = W // TCC
LUT_CH = DIM ** 3
LUT_PAD = 36096

_INV_BINSIZE = jnp.float32((DIM - 1) / 1.000001)

UNROLL = 8


def _compute_chunk(lut_v, in_b, out_b):
    himask = jnp.int32(-65536)

    def step(rr, o):
        r = in_b[0][rr, pl.ds(o, L)]
        g = in_b[1][rr, pl.ds(o, L)]
        b = in_b[2][rr, pl.ds(o, L)]
        tr = r * _INV_BINSIZE
        tg = g * _INV_BINSIZE
        tb = b * _INV_BINSIZE
        rid = tr.astype(jnp.int32)
        gid = tg.astype(jnp.int32)
        bid = tb.astype(jnp.int32)
        rd = tr - rid.astype(jnp.float32)
        gd = tg - gid.astype(jnp.float32)
        bd = tb - bid.astype(jnp.float32)
        id000 = rid + gid * DIM + bid * (DIM * DIM)
        pidx = (id000, id000 + DIM, id000 + DIM * DIM,
                id000 + DIM * DIM + DIM)
        for ch in range(3):
            v = []
            for k in range(4):
                wv = plsc.load_gather(lut_v[ch], [pidx[k]])
                lo = plsc.bitcast(wv << 16, jnp.float32)
                hi = plsc.bitcast(wv & himask, jnp.float32)
                v.append(lo + rd * (hi - lo))
            vb0 = v[0] + gd * (v[1] - v[0])
            vb1 = v[2] + gd * (v[3] - v[2])
            out_b[ch][rr, pl.ds(o, L)] = vb0 + bd * (vb1 - vb0)

    def inner(i, carry):
        for u in range(UNROLL):
            step(i, u * L)
        return 0

    lax.fori_loop(0, TR, inner, 0)


def _body(x_hbm, lut_hbm, out_hbm, *rest):
    lut_v = rest[:3]
    bufs, sems = rest[3:15], rest[15:]
    in_b = (bufs[0:3], bufs[3:6])
    out_b = (bufs[6:9], bufs[9:12])
    lsem = sems[0:2]
    ssem = sems[2:4]
    wid = lax.axis_index("s") * NC + lax.axis_index("c")
    for ch in range(3):
        pltpu.sync_copy(lut_hbm.at[pl.ds(ch * LUT_PAD, LUT_PAD)], lut_v[ch])

    def tile_slices(t):
        bi = lax.shift_right_logical(t, 3)
        j = lax.bitwise_and(t, TPW - 1)
        g = wid * TPW + j
        rb = lax.shift_right_logical(g, 2)
        cb = lax.bitwise_and(g, CBLK - 1)
        return bi, rb * TR, cb * TCC

    def src(t, ch):
        bi, r0, c0 = tile_slices(t)
        return x_hbm.at[bi, ch, pl.ds(r0, TR), pl.ds(c0, TCC)]

    def dst(t, ch):
        bi, r0, c0 = tile_slices(t)
        return out_hbm.at[bi, ch, pl.ds(r0, TR), pl.ds(c0, TCC)]

    for tt in (0, 1):
        for ch in range(3):
            pltpu.async_copy(src(jnp.int32(tt), ch), in_b[tt][ch], lsem[tt])

    def pair(p, carry):
        for s in (0, 1):
            t = 2 * p + s
            for ch in range(3):
                pltpu.make_async_copy(src(t, ch), in_b[s][ch],
                                      lsem[s]).wait()

            @pl.when(p >= 1)
            def _():
                for ch in range(3):
                    pltpu.make_async_copy(out_b[s][ch], dst(jnp.int32(0), ch),
                                          ssem[s]).wait()

            _compute_chunk(lut_v, in_b[s], out_b[s])
            for ch in range(3):
                pltpu.async_copy(out_b[s][ch], dst(t, ch), ssem[s])

            @pl.when(p < (NT // 2) - 1)
            def _():
                for ch in range(3):
                    pltpu.async_copy(src(t + 2, ch), in_b[s][ch], lsem[s])
        return 0

    lax.fori_loop(0, NT // 2, pair, 0)
    for s in (0, 1):
        for ch in range(3):
            pltpu.make_async_copy(out_b[s][ch], dst(jnp.int32(0), ch),
                                  ssem[s]).wait()


@jax.jit
def kernel(x, lut):
    lb = jax.lax.bitcast_convert_type(
        lut.reshape(3, DIM ** 3).astype(jnp.bfloat16), jnp.uint16
    ).astype(jnp.uint32)
    nxt = jnp.concatenate([lb[:, 1:], lb[:, :1]], axis=1)
    packed = jax.lax.bitcast_convert_type(lb | (nxt << 16), jnp.int32)
    lutf = jnp.pad(packed, ((0, 0), (0, LUT_PAD - LUT_CH))).reshape(3 * LUT_PAD)
    mesh = plsc.VectorSubcoreMesh(core_axis_name="c", subcore_axis_name="s",
                                  num_cores=NC, num_subcores=NS)
    out = pl.kernel(
        _body,
        out_type=jax.ShapeDtypeStruct((B, 3, H, W), jnp.float32),
        mesh=mesh,
        scratch_types=[pltpu.VMEM((LUT_PAD,), jnp.int32) for _ in range(3)]
        + [pltpu.VMEM((TR, TCC), jnp.float32) for _ in range(12)]
        + [pltpu.SemaphoreType.DMA for _ in range(4)],
        compiler_params=pltpu.CompilerParams(needs_layout_passes=False,
                                             use_tc_tiling_on_sc=True),
    )(x, lutf)
    return out


# speedup vs baseline: 1.3887x; 1.3887x over previous
import functools

import jax
import jax.numpy as jnp
from jax import lax
from jax.experimental import pallas as pl
from jax.experimental.pallas import tpu as pltpu
from jax.experimental.pallas import tpu_sc as plsc

DIM = 33
NC, NS, L = 2, 16, 16
NW = NC * NS
B, H, W = 8, 512, 512
TR, TCC = 8, 128
TPB = (H // TR) * (W // TCC)
TPW = TPB // NW
NT = B * TPW
CBLK = W // TCC
LUT_CH = DIM ** 3
LUT_PAD = 36096

_INV_BINSIZE = jnp.float32((DIM - 1) / 1.000001)

UNROLL = 8


def _compute_chunk(lut_v, in_b, out_b):
    himask = jnp.int32(-65536)

    def step(rr, o):
        r = in_b[0][rr, pl.ds(o, L)]
        g = in_b[1][rr, pl.ds(o, L)]
        b = in_b[2][rr, pl.ds(o, L)]
        tr = r * _INV_BINSIZE
        tg = g * _INV_BINSIZE
        tb = b * _INV_BINSIZE
        rid = tr.astype(jnp.int32)
        gid = tg.astype(jnp.int32)
        bid = tb.astype(jnp.int32)
        rd = tr - rid.astype(jnp.float32)
        gd = tg - gid.astype(jnp.float32)
        bd = tb - bid.astype(jnp.float32)
        one = jnp.float32(1.0)
        omr, omg, omb = one - rd, one - gd, one - bd
        w00, w10, w01, w11 = omr * omg, rd * omg, omr * gd, rd * gd
        w = ((w00 * omb, w10 * omb), (w01 * omb, w11 * omb),
             (w00 * bd, w10 * bd), (w01 * bd, w11 * bd))
        id000 = rid + gid * DIM + bid * (DIM * DIM)
        pidx = (id000, id000 + DIM, id000 + DIM * DIM,
                id000 + DIM * DIM + DIM)
        for ch in range(3):
            acc = None
            for k in range(4):
                wv = plsc.load_gather(lut_v[ch], [pidx[k]])
                lo = plsc.bitcast(wv << 16, jnp.float32)
                hi = plsc.bitcast(wv & himask, jnp.float32)
                term = w[k][0] * lo + w[k][1] * hi
                acc = term if acc is None else acc + term
            out_b[ch][rr, pl.ds(o, L)] = acc

    def inner(i, carry):
        for u in range(UNROLL):
            step(i, u * L)
        return 0

    lax.fori_loop(0, TR, inner, 0)


def _body(x_hbm, lut_hbm, out_hbm, *rest):
    lut_v = rest[:3]
    bufs, sems = rest[3:15], rest[15:]
    in_b = (bufs[0:3], bufs[3:6])
    out_b = (bufs[6:9], bufs[9:12])
    lsem = sems[0:2]
    ssem = sems[2:4]
    wid = lax.axis_index("s") * NC + lax.axis_index("c")
    for ch in range(3):
        pltpu.sync_copy(lut_hbm.at[pl.ds(ch * LUT_PAD, LUT_PAD)], lut_v[ch])

    def tile_slices(t):
        bi = lax.shift_right_logical(t, 3)
        j = lax.bitwise_and(t, TPW - 1)
        g = wid * TPW + j
        rb = lax.shift_right_logical(g, 2)
        cb = lax.bitwise_and(g, CBLK - 1)
        return bi, rb * TR, cb * TCC

    def src(t, ch):
        bi, r0, c0 = tile_slices(t)
        return x_hbm.at[bi, ch, pl.ds(r0, TR), pl.ds(c0, TCC)]

    def dst(t, ch):
        bi, r0, c0 = tile_slices(t)
        return out_hbm.at[bi, ch, pl.ds(r0, TR), pl.ds(c0, TCC)]

    for tt in (0, 1):
        for ch in range(3):
            pltpu.async_copy(src(jnp.int32(tt), ch), in_b[tt][ch], lsem[tt])

    def pair(p, carry):
        for s in (0, 1):
            t = 2 * p + s
            for ch in range(3):
                pltpu.make_async_copy(src(t, ch), in_b[s][ch],
                                      lsem[s]).wait()

            @pl.when(p >= 1)
            def _():
                for ch in range(3):
                    pltpu.make_async_copy(out_b[s][ch], dst(jnp.int32(0), ch),
                                          ssem[s]).wait()

            _compute_chunk(lut_v, in_b[s], out_b[s])
            for ch in range(3):
                pltpu.async_copy(out_b[s][ch], dst(t, ch), ssem[s])

            @pl.when(p < (NT // 2) - 1)
            def _():
                for ch in range(3):
                    pltpu.async_copy(src(t + 2, ch), in_b[s][ch], lsem[s])
        return 0

    lax.fori_loop(0, NT // 2, pair, 0)
    for s in (0, 1):
        for ch in range(3):
            pltpu.make_async_copy(out_b[s][ch], dst(jnp.int32(0), ch),
                                  ssem[s]).wait()


@jax.jit
def kernel(x, lut):
    lb = jax.lax.bitcast_convert_type(
        lut.reshape(3, DIM ** 3).astype(jnp.bfloat16), jnp.uint16
    ).astype(jnp.uint32)
    nxt = jnp.concatenate([lb[:, 1:], lb[:, :1]], axis=1)
    packed = jax.lax.bitcast_convert_type(lb | (nxt << 16), jnp.int32)
    lutf = jnp.pad(packed, ((0, 0), (0, LUT_PAD - LUT_CH))).reshape(3 * LUT_PAD)
    mesh = plsc.VectorSubcoreMesh(core_axis_name="c", subcore_axis_name="s",
                                  num_cores=NC, num_subcores=NS)
    out = pl.kernel(
        _body,
        out_type=jax.ShapeDtypeStruct((B, 3, H, W), jnp.float32),
        mesh=mesh,
        scratch_types=[pltpu.VMEM((LUT_PAD,), jnp.int32) for _ in range(3)]
        + [pltpu.VMEM((TR, TCC), jnp.float32) for _ in range(12)]
        + [pltpu.SemaphoreType.DMA for _ in range(4)],
        compiler_params=pltpu.CompilerParams(needs_layout_passes=False,
                                             use_tc_tiling_on_sc=True),
    )(x, lutf)
    return out
